# baseline (speedup 1.0000x reference)
import jax
import jax.numpy as jnp
from jax.experimental import pallas as pl
from jax.experimental.pallas import tpu as pltpu


def _palette_quant_body(x_ref, m_ref, a_ref, o_ref):
    x = x_ref[0]
    m = m_ref[0]
    a = a_ref[0]

    n = x.shape[0]
    ones = jnp.ones((n, 1), dtype=x.dtype)
    xaug = jnp.concatenate([x, ones], axis=1)
    t = jnp.dot(xaug, m, preferred_element_type=jnp.float32)
    e = jnp.exp(t)
    r = jnp.dot(e, a, preferred_element_type=jnp.float32)
    inv = 1.0 / r[:, 3:4]
    o_ref[0] = r[:, 0:3] * inv


def kernel(images, palettes, temperature):
    B, H, W, C = images.shape
    K = palettes.shape[1]
    HW = H * W
    N = 2048
    grid = (B, HW // N)

    x = images.reshape(B, HW, C)
    scale = 2.0 / temperature
    bias = -jnp.sum(palettes * palettes, axis=-1) / temperature
    m = jnp.concatenate(
        [palettes.transpose(0, 2, 1) * scale, bias[:, None, :]], axis=1
    )
    a = jnp.concatenate(
        [palettes, jnp.ones((B, K, 1), palettes.dtype)], axis=-1
    )

    out = pl.pallas_call(
        _palette_quant_body,
        grid=grid,
        in_specs=[
            pl.BlockSpec((1, N, C), lambda i, j: (i, j, 0)),
            pl.BlockSpec((1, C + 1, K), lambda i, j: (i, 0, 0)),
            pl.BlockSpec((1, K, C + 1), lambda i, j: (i, 0, 0)),
        ],
        out_specs=pl.BlockSpec((1, N, C), lambda i, j: (i, j, 0)),
        out_shape=jax.ShapeDtypeStruct((B, HW, C), jnp.float32),
    )(x, m, a)

    return out.reshape(B, H, W, C)


# speedup vs baseline: 4.7955x; 4.7955x over previous
import jax
import jax.numpy as jnp
from jax.experimental import pallas as pl
from jax.experimental.pallas import tpu as pltpu


def _palette_quant_body(x_ref, m_ref, a_ref, o_ref):
    x = x_ref[0]
    m = m_ref[0]
    a = a_ref[0]

    n = x.shape[1]
    ones = jnp.ones((1, n), dtype=x.dtype)
    xaug = jnp.concatenate([x, ones], axis=0)
    t = jnp.dot(m, xaug, preferred_element_type=jnp.float32)
    e = jnp.exp(t)
    r = jnp.dot(a, e, preferred_element_type=jnp.float32)
    inv = 1.0 / r[3:4, :]
    o_ref[0] = r[0:3, :] * inv


def kernel(images, palettes, temperature):
    B, H, W, C = images.shape
    K = palettes.shape[1]
    HW = H * W
    N = 2048
    grid = (B, HW // N)

    xp = images.reshape(B, HW, C).transpose(0, 2, 1)
    scale = 2.0 / temperature
    bias = -jnp.sum(palettes * palettes, axis=-1) / temperature
    m = jnp.concatenate([palettes * scale, bias[..., None]], axis=-1)
    a = jnp.concatenate(
        [palettes, jnp.ones((B, K, 1), palettes.dtype)], axis=-1
    ).transpose(0, 2, 1)

    out_planar = pl.pallas_call(
        _palette_quant_body,
        grid=grid,
        in_specs=[
            pl.BlockSpec((1, C, N), lambda i, j: (i, 0, j)),
            pl.BlockSpec((1, K, C + 1), lambda i, j: (i, 0, 0)),
            pl.BlockSpec((1, C + 1, K), lambda i, j: (i, 0, 0)),
        ],
        out_specs=pl.BlockSpec((1, C, N), lambda i, j: (i, 0, j)),
        out_shape=jax.ShapeDtypeStruct((B, C, HW), jnp.float32),
    )(xp, m, a)

    return out_planar.transpose(0, 2, 1).reshape(B, H, W, C)
